# baseline (speedup 1.0000x reference)
import functools

import jax
import jax.numpy as jnp
from jax import lax
from jax.experimental import pallas as pl
from jax.experimental.pallas import tpu as pltpu
from jax.experimental.pallas import tpu_sc as plsc

EMBED_DIM = 2048
TEXT_DIM = 768
NUM_EXPERTS = 64
TOP_K = 8
BATCH = 4
SEQ = 4096

SC_TOKENS = 1024
TC_TOKENS = SEQ - SC_TOKENS
TC_CHUNK = 512
N_TC_CHUNK = TC_TOKENS // TC_CHUNK

N_WORKERS = 32
DSLICES = N_WORKERS // BATCH
DSLICE = EMBED_DIM // DSLICES
SC_BLK = 128
N_SC_BLK = SC_TOKENS // SC_BLK
LANES = 16
VREGS = DSLICE // LANES


def _sc_sum_body(x_hbm, out_hbm, buf, accv, sem0, sem1):
    c = lax.axis_index("c")
    s = lax.axis_index("s")
    wid = s * 2 + c
    b = wid // DSLICES
    d0 = (wid % DSLICES) * DSLICE

    sems = (sem0, sem1)

    def blk_src(i):
        return x_hbm.at[b, pl.ds(TC_TOKENS + i * SC_BLK, SC_BLK),
                        pl.ds(d0, DSLICE)]

    copies = [None] * N_SC_BLK
    copies[0] = pltpu.async_copy(blk_src(0), buf.at[0], sems[0])

    accs = tuple(jnp.zeros((LANES,), jnp.float32) for _ in range(VREGS))
    for i in range(N_SC_BLK):
        if i + 1 < N_SC_BLK:
            copies[i + 1] = pltpu.async_copy(
                blk_src(i + 1), buf.at[(i + 1) % 2], sems[(i + 1) % 2])
        copies[i].wait()
        cur = buf.at[i % 2]

        def token_body(r, acc):
            return tuple(acc[l] + cur[r, pl.ds(l * LANES, LANES)]
                         for l in range(VREGS))

        accs = lax.fori_loop(0, SC_BLK, token_body, accs)

    for l in range(VREGS):
        accv[pl.ds(l * LANES, LANES)] = accs[l]
    pltpu.sync_copy(accv, out_hbm.at[b, pl.ds(d0, DSLICE)])


@functools.cache
def _sc_token_sum():
    return pl.kernel(
        _sc_sum_body,
        mesh=plsc.VectorSubcoreMesh(core_axis_name="c", subcore_axis_name="s"),
        out_type=jax.ShapeDtypeStruct((BATCH, EMBED_DIM), jnp.float32),
        scratch_types=[
            pltpu.VMEM((2, SC_BLK, DSLICE), jnp.float32),
            pltpu.VMEM((DSLICE,), jnp.float32),
            pltpu.SemaphoreType.DMA,
            pltpu.SemaphoreType.DMA,
        ],
    )


def _tc_sum_kernel(x_ref, out_ref, acc_ref):
    step = pl.program_id(0)
    partial = jnp.sum(x_ref[...], axis=1)

    @pl.when(step == 0)
    def _init():
        acc_ref[...] = partial

    @pl.when(step != 0)
    def _accum():
        acc_ref[...] = acc_ref[...] + partial

    @pl.when(step == N_TC_CHUNK - 1)
    def _emit():
        out_ref[...] = acc_ref[...]


W1_BLK = 256
N_W1_BLK = (EMBED_DIM + TEXT_DIM) // W1_BLK


def _combine_kernel(tc_ref, sc_ref, t_ref, w1_ref, b1_ref, w2_ref, b2_ref,
                    w_out_ref, i_out_ref, df_ref, hacc_ref):
    step = pl.program_id(0)

    @pl.when(step == 0)
    def _init():
        df_ref[:, :EMBED_DIM] = (tc_ref[...] + sc_ref[...]) * (1.0 / SEQ)
        df_ref[:, EMBED_DIM:] = t_ref[...]

    k0 = pl.multiple_of(step * W1_BLK, W1_BLK)
    part = jnp.dot(df_ref[:, pl.ds(k0, W1_BLK)], w1_ref[...],
                   preferred_element_type=jnp.float32)

    @pl.when(step == 0)
    def _hinit():
        hacc_ref[...] = part

    @pl.when(step != 0)
    def _haccum():
        hacc_ref[...] = hacc_ref[...] + part

    @pl.when(step == N_W1_BLK - 1)
    def _finish():
        h = jnp.maximum(hacc_ref[...] + b1_ref[...], 0.0)
        logits = (jnp.dot(h, w2_ref[...], preferred_element_type=jnp.float32)
                  + b2_ref[...])

        iota = lax.broadcasted_iota(jnp.int32, (BATCH, NUM_EXPERTS), 1)
        cur = logits
        vals = []
        idxs = []
        for _ in range(TOP_K):
            m = jnp.max(cur, axis=1, keepdims=True)
            sel = cur == m
            idx = jnp.min(jnp.where(sel, iota, NUM_EXPERTS),
                          axis=1, keepdims=True)
            vals.append(m)
            idxs.append(idx)
            cur = jnp.where(iota == idx, -jnp.inf, cur)
        top_v = jnp.concatenate(vals, axis=1)
        top_i = jnp.concatenate(idxs, axis=1)
        e = jnp.exp(top_v - top_v[:, 0:1])
        w = e / jnp.sum(e, axis=1, keepdims=True)
        w_out_ref[...] = w
        i_out_ref[...] = top_i


@jax.jit
def kernel(x, text_embedding, W1, b1, W2, b2):
    tc_sum = pl.pallas_call(
        _tc_sum_kernel,
        grid=(N_TC_CHUNK,),
        in_specs=[pl.BlockSpec((BATCH, TC_CHUNK, EMBED_DIM),
                               lambda i: (0, i, 0))],
        out_specs=pl.BlockSpec((BATCH, EMBED_DIM), lambda i: (0, 0)),
        out_shape=jax.ShapeDtypeStruct((BATCH, EMBED_DIM), jnp.float32),
        scratch_shapes=[pltpu.VMEM((BATCH, EMBED_DIM), jnp.float32)],
        compiler_params=pltpu.CompilerParams(
            dimension_semantics=("arbitrary",),
        ),
    )(x)

    sc_sum = _sc_token_sum()(x)

    b1r = b1.reshape(1, EMBED_DIM)
    b2r = b2.reshape(1, NUM_EXPERTS)
    out = pl.pallas_call(
        _combine_kernel,
        grid=(N_W1_BLK,),
        in_specs=[
            pl.BlockSpec((BATCH, EMBED_DIM), lambda i: (0, 0)),
            pl.BlockSpec((BATCH, EMBED_DIM), lambda i: (0, 0)),
            pl.BlockSpec((BATCH, TEXT_DIM), lambda i: (0, 0)),
            pl.BlockSpec((W1_BLK, EMBED_DIM), lambda i: (i, 0)),
            pl.BlockSpec((1, EMBED_DIM), lambda i: (0, 0)),
            pl.BlockSpec((EMBED_DIM, NUM_EXPERTS), lambda i: (0, 0)),
            pl.BlockSpec((1, NUM_EXPERTS), lambda i: (0, 0)),
        ],
        out_specs=[
            pl.BlockSpec((BATCH, TOP_K), lambda i: (0, 0)),
            pl.BlockSpec((BATCH, TOP_K), lambda i: (0, 0)),
        ],
        out_shape=[
            jax.ShapeDtypeStruct((BATCH, TOP_K), jnp.float32),
            jax.ShapeDtypeStruct((BATCH, TOP_K), jnp.int32),
        ],
        scratch_shapes=[pltpu.VMEM((BATCH, EMBED_DIM + TEXT_DIM), jnp.float32),
                        pltpu.VMEM((BATCH, EMBED_DIM), jnp.float32)],
        compiler_params=pltpu.CompilerParams(
            dimension_semantics=("arbitrary",),
        ),
    )(tc_sum, sc_sum, text_embedding, W1, b1r, W2, b2r)
    return (out[0], out[1])


# speedup vs baseline: 1.3001x; 1.3001x over previous
import jax
import jax.numpy as jnp
from jax import lax
from jax.experimental import pallas as pl
from jax.experimental.pallas import tpu as pltpu

EMBED_DIM = 2048
TEXT_DIM = 768
NUM_EXPERTS = 64
TOP_K = 8
BATCH = 4
SEQ = 4096

CHUNK = 256
NSTEP = SEQ // CHUNK
NBUF = 4


def _gating_kernel(x_ref, t_ref, w1_ref, b1_ref, w2_ref, b2_ref,
                   w_out_ref, i_out_ref, xbuf_ref, sem):
    copies = [
        pltpu.make_async_copy(
            x_ref.at[:, pl.ds(i * CHUNK, CHUNK), :],
            xbuf_ref.at[i % NBUF],
            sem.at[i % NBUF],
        )
        for i in range(NSTEP)
    ]
    for i in range(NBUF):
        copies[i].start()

    acc = jnp.zeros((BATCH, EMBED_DIM), jnp.float32)
    for i in range(NSTEP):
        copies[i].wait()
        acc = acc + jnp.sum(xbuf_ref[i % NBUF], axis=1)
        if i + NBUF < NSTEP:
            copies[i + NBUF].start()

    x_mean = acc * (1.0 / SEQ)
    h = (jnp.dot(x_mean, w1_ref[:EMBED_DIM, :],
                 preferred_element_type=jnp.float32)
         + jnp.dot(t_ref[...], w1_ref[EMBED_DIM:, :],
                   preferred_element_type=jnp.float32)
         + b1_ref[...])
    h = jnp.maximum(h, 0.0)
    logits = (jnp.dot(h, w2_ref[...], preferred_element_type=jnp.float32)
              + b2_ref[...])

    iota = lax.broadcasted_iota(jnp.int32, (BATCH, NUM_EXPERTS), 1)
    cur = logits
    vals = []
    idxs = []
    for _ in range(TOP_K):
        m = jnp.max(cur, axis=1, keepdims=True)
        sel = cur == m
        idx = jnp.min(jnp.where(sel, iota, NUM_EXPERTS),
                      axis=1, keepdims=True)
        vals.append(m)
        idxs.append(idx)
        cur = jnp.where(iota == idx, -jnp.inf, cur)
    top_v = jnp.concatenate(vals, axis=1)
    top_i = jnp.concatenate(idxs, axis=1)
    e = jnp.exp(top_v - top_v[:, 0:1])
    w = e / jnp.sum(e, axis=1, keepdims=True)
    w_out_ref[...] = w
    i_out_ref[...] = top_i


@jax.jit
def kernel(x, text_embedding, W1, b1, W2, b2):
    b1r = b1.reshape(1, EMBED_DIM)
    b2r = b2.reshape(1, NUM_EXPERTS)
    out = pl.pallas_call(
        _gating_kernel,
        in_specs=[
            pl.BlockSpec(memory_space=pl.ANY),
            pl.BlockSpec((BATCH, TEXT_DIM), lambda: (0, 0)),
            pl.BlockSpec((EMBED_DIM + TEXT_DIM, EMBED_DIM), lambda: (0, 0)),
            pl.BlockSpec((1, EMBED_DIM), lambda: (0, 0)),
            pl.BlockSpec((EMBED_DIM, NUM_EXPERTS), lambda: (0, 0)),
            pl.BlockSpec((1, NUM_EXPERTS), lambda: (0, 0)),
        ],
        out_specs=[
            pl.BlockSpec((BATCH, TOP_K), lambda: (0, 0)),
            pl.BlockSpec((BATCH, TOP_K), lambda: (0, 0)),
        ],
        out_shape=[
            jax.ShapeDtypeStruct((BATCH, TOP_K), jnp.float32),
            jax.ShapeDtypeStruct((BATCH, TOP_K), jnp.int32),
        ],
        scratch_shapes=[
            pltpu.VMEM((NBUF, BATCH, CHUNK, EMBED_DIM), jnp.float32),
            pltpu.SemaphoreType.DMA((NBUF,)),
        ],
    )(x, text_embedding, W1, b1r, W2, b2r)
    return (out[0], out[1])


# speedup vs baseline: 1.3114x; 1.0087x over previous
import jax
import jax.numpy as jnp
from jax import lax
from jax.experimental import pallas as pl
from jax.experimental.pallas import tpu as pltpu

EMBED_DIM = 2048
TEXT_DIM = 768
NUM_EXPERTS = 64
TOP_K = 8
BATCH = 4
SEQ = 4096

CHUNK = 512
NCHUNK = SEQ // CHUNK


def _gating_kernel(x_ref, t_ref, w1_ref, b1_ref, w2_ref, b2_ref,
                   w_out_ref, i_out_ref, sums_ref):
    b = pl.program_id(0)
    step = pl.program_id(1)

    partial = jnp.sum(x_ref[...], axis=1)

    @pl.when(step == 0)
    def _init():
        sums_ref[pl.ds(b, 1)] = partial

    @pl.when(step != 0)
    def _accum():
        sums_ref[pl.ds(b, 1)] = sums_ref[pl.ds(b, 1)] + partial

    @pl.when((b == BATCH - 1) & (step == NCHUNK - 1))
    def _finish():
        x_mean = sums_ref[...] * (1.0 / SEQ)
        h = (jnp.dot(x_mean, w1_ref[:EMBED_DIM, :],
                     preferred_element_type=jnp.float32)
             + jnp.dot(t_ref[...], w1_ref[EMBED_DIM:, :],
                       preferred_element_type=jnp.float32)
             + b1_ref[...])
        h = jnp.maximum(h, 0.0)
        logits = (jnp.dot(h, w2_ref[...], preferred_element_type=jnp.float32)
                  + b2_ref[...])

        iota = lax.broadcasted_iota(jnp.int32, (BATCH, NUM_EXPERTS), 1)
        cur = logits
        vals = []
        idxs = []
        for _ in range(TOP_K):
            m = jnp.max(cur, axis=1, keepdims=True)
            sel = cur == m
            idx = jnp.min(jnp.where(sel, iota, NUM_EXPERTS),
                          axis=1, keepdims=True)
            vals.append(m)
            idxs.append(idx)
            cur = jnp.where(iota == idx, -jnp.inf, cur)
        top_v = jnp.concatenate(vals, axis=1)
        top_i = jnp.concatenate(idxs, axis=1)
        e = jnp.exp(top_v - top_v[:, 0:1])
        w = e / jnp.sum(e, axis=1, keepdims=True)
        w_out_ref[...] = w
        i_out_ref[...] = top_i


@jax.jit
def kernel(x, text_embedding, W1, b1, W2, b2):
    b1r = b1.reshape(1, EMBED_DIM)
    b2r = b2.reshape(1, NUM_EXPERTS)
    out = pl.pallas_call(
        _gating_kernel,
        grid=(BATCH, NCHUNK),
        in_specs=[
            pl.BlockSpec((1, CHUNK, EMBED_DIM), lambda b, i: (b, i, 0)),
            pl.BlockSpec((BATCH, TEXT_DIM), lambda b, i: (0, 0)),
            pl.BlockSpec((EMBED_DIM + TEXT_DIM, EMBED_DIM),
                         lambda b, i: (0, 0)),
            pl.BlockSpec((1, EMBED_DIM), lambda b, i: (0, 0)),
            pl.BlockSpec((EMBED_DIM, NUM_EXPERTS), lambda b, i: (0, 0)),
            pl.BlockSpec((1, NUM_EXPERTS), lambda b, i: (0, 0)),
        ],
        out_specs=[
            pl.BlockSpec((BATCH, TOP_K), lambda b, i: (0, 0)),
            pl.BlockSpec((BATCH, TOP_K), lambda b, i: (0, 0)),
        ],
        out_shape=[
            jax.ShapeDtypeStruct((BATCH, TOP_K), jnp.float32),
            jax.ShapeDtypeStruct((BATCH, TOP_K), jnp.int32),
        ],
        scratch_shapes=[pltpu.VMEM((BATCH, EMBED_DIM), jnp.float32)],
        compiler_params=pltpu.CompilerParams(
            dimension_semantics=("arbitrary", "arbitrary"),
        ),
    )(x, text_embedding, W1, b1r, W2, b2r)
    return (out[0], out[1])


# speedup vs baseline: 1.3768x; 1.0499x over previous
import jax
import jax.numpy as jnp
from jax import lax
from jax.experimental import pallas as pl
from jax.experimental.pallas import tpu as pltpu

EMBED_DIM = 2048
TEXT_DIM = 768
NUM_EXPERTS = 64
TOP_K = 8
BATCH = 4
SEQ = 4096

CHUNK = 1024
NCHUNK = SEQ // CHUNK


def _gating_kernel(x_ref, t_ref, w1_ref, b1_ref, w2_ref, b2_ref,
                   w_out_ref, i_out_ref, sums_ref):
    b = pl.program_id(0)
    step = pl.program_id(1)

    partial = jnp.sum(x_ref[...], axis=1)

    @pl.when(step == 0)
    def _init():
        sums_ref[pl.ds(b, 1)] = partial

    @pl.when(step != 0)
    def _accum():
        sums_ref[pl.ds(b, 1)] = sums_ref[pl.ds(b, 1)] + partial

    @pl.when((b == BATCH - 1) & (step == NCHUNK - 1))
    def _finish():
        x_mean = sums_ref[...] * (1.0 / SEQ)
        h = (jnp.dot(x_mean, w1_ref[:EMBED_DIM, :],
                     preferred_element_type=jnp.float32)
             + jnp.dot(t_ref[...], w1_ref[EMBED_DIM:, :],
                       preferred_element_type=jnp.float32)
             + b1_ref[...])
        h = jnp.maximum(h, 0.0)
        logits = (jnp.dot(h, w2_ref[...], preferred_element_type=jnp.float32)
                  + b2_ref[...])

        iota = lax.broadcasted_iota(jnp.int32, (BATCH, NUM_EXPERTS), 1)
        cur = logits
        vals = []
        idxs = []
        for _ in range(TOP_K):
            m = jnp.max(cur, axis=1, keepdims=True)
            sel = cur == m
            idx = jnp.min(jnp.where(sel, iota, NUM_EXPERTS),
                          axis=1, keepdims=True)
            vals.append(m)
            idxs.append(idx)
            cur = jnp.where(iota == idx, -jnp.inf, cur)
        top_v = jnp.concatenate(vals, axis=1)
        top_i = jnp.concatenate(idxs, axis=1)
        e = jnp.exp(top_v - top_v[:, 0:1])
        w = e / jnp.sum(e, axis=1, keepdims=True)
        w_out_ref[...] = w
        i_out_ref[...] = top_i


@jax.jit
def kernel(x, text_embedding, W1, b1, W2, b2):
    b1r = b1.reshape(1, EMBED_DIM)
    b2r = b2.reshape(1, NUM_EXPERTS)
    out = pl.pallas_call(
        _gating_kernel,
        grid=(BATCH, NCHUNK),
        in_specs=[
            pl.BlockSpec((1, CHUNK, EMBED_DIM), lambda b, i: (b, i, 0)),
            pl.BlockSpec((BATCH, TEXT_DIM), lambda b, i: (0, 0)),
            pl.BlockSpec((EMBED_DIM + TEXT_DIM, EMBED_DIM),
                         lambda b, i: (0, 0)),
            pl.BlockSpec((1, EMBED_DIM), lambda b, i: (0, 0)),
            pl.BlockSpec((EMBED_DIM, NUM_EXPERTS), lambda b, i: (0, 0)),
            pl.BlockSpec((1, NUM_EXPERTS), lambda b, i: (0, 0)),
        ],
        out_specs=[
            pl.BlockSpec((BATCH, TOP_K), lambda b, i: (0, 0)),
            pl.BlockSpec((BATCH, TOP_K), lambda b, i: (0, 0)),
        ],
        out_shape=[
            jax.ShapeDtypeStruct((BATCH, TOP_K), jnp.float32),
            jax.ShapeDtypeStruct((BATCH, TOP_K), jnp.int32),
        ],
        scratch_shapes=[pltpu.VMEM((BATCH, EMBED_DIM), jnp.float32)],
        compiler_params=pltpu.CompilerParams(
            dimension_semantics=("arbitrary", "arbitrary"),
            vmem_limit_bytes=120 * 1024 * 1024,
        ),
    )(x, text_embedding, W1, b1r, W2, b2r)
    return (out[0], out[1])
